# Initial kernel scaffold; baseline (speedup 1.0000x reference)
import jax
import jax.numpy as jnp
from jax.experimental import pallas as pl


def kernel(X, edge_index, W1, b1, W2, b2):
    raise NotImplementedError("write your pallas kernel here")


# speedup vs baseline: 23.1641x; 23.1641x over previous
import functools

import jax
import jax.numpy as jnp
from jax import lax
from jax.experimental import pallas as pl
from jax.experimental.pallas import tpu as pltpu
from jax.experimental.pallas import tpu_sc as plsc

N = 10000
F = 128
HID = 512
CPAD = 16
N_PAD = 10240
E = 320000
NCORES = 2
NSUB = 16
NTILES = NCORES * NSUB
BATCH = 128
NBATCH = 79
EPT = NBATCH * BATCH
E_PAD = NTILES * EPT
RPT = N_PAD // NSUB
ZROWS = N_PAD - N


def _mesh():
  return plsc.VectorSubcoreMesh(
      core_axis_name="c", subcore_axis_name="s",
      num_cores=NCORES, num_subcores=NSUB)


def _sc_degree(dst_idx):
  @functools.partial(
      pl.kernel,
      out_type=jax.ShapeDtypeStruct((NCORES, N_PAD), jnp.float32),
      mesh=_mesh(),
      compiler_params=pltpu.CompilerParams(use_tc_tiling_on_sc=False),
      scratch_types=[
          pltpu.VMEM((NBATCH, BATCH), jnp.int32),
          pltpu.VMEM((BATCH,), jnp.float32),
          pltpu.VMEM((RPT,), jnp.float32),
          pltpu.VMEM_SHARED((N_PAD,), jnp.float32),
      ],
  )
  def deg_kernel(dst_hbm, out_hbm, dstv, ones_v, zb, acc):
    c = lax.axis_index("c")
    s = lax.axis_index("s")
    w = c * NSUB + s
    pltpu.sync_copy(dst_hbm.at[w], dstv)
    for i in range(BATCH // 16):
      ones_v[pl.ds(i * 16, 16)] = jnp.ones((16,), jnp.float32)
    for i in range(RPT // 16):
      zb[pl.ds(i * 16, 16)] = jnp.zeros((16,), jnp.float32)
    pltpu.sync_copy(zb, acc.at[pl.ds(s * RPT, RPT)])
    plsc.subcore_barrier()

    def step(j, carry):
      pltpu.sync_copy(ones_v, acc.at[dstv.at[j]], add=True)
      return carry

    lax.fori_loop(0, NBATCH, step, None)
    plsc.subcore_barrier()
    pltpu.sync_copy(acc.at[pl.ds(s * RPT, RPT)],
                    out_hbm.at[c, pl.ds(s * RPT, RPT)])

  return deg_kernel(dst_idx)


def _sc_seg_sum(values, src_idx, dst_idx, width):
  params = (None if width % 128 == 0
            else pltpu.CompilerParams(use_tc_tiling_on_sc=False))

  @functools.partial(
      pl.kernel,
      out_type=jax.ShapeDtypeStruct((NCORES, N_PAD, width), jnp.float32),
      mesh=_mesh(),
      compiler_params=params,
      scratch_types=[
          pltpu.VMEM((NBATCH, BATCH), jnp.int32),
          pltpu.VMEM((NBATCH, BATCH), jnp.int32),
          pltpu.VMEM((BATCH, width), jnp.float32),
          pltpu.VMEM_SHARED((N_PAD, width), jnp.float32),
      ],
  )
  def seg_kernel(vals_hbm, src_hbm, dst_hbm, out_hbm, srcv, dstv, rows, acc):
    c = lax.axis_index("c")
    s = lax.axis_index("s")
    w = c * NSUB + s
    pltpu.sync_copy(src_hbm.at[w], srcv)
    pltpu.sync_copy(dst_hbm.at[w], dstv)
    base = s * RPT
    pltpu.sync_copy(vals_hbm.at[pl.ds(N, ZROWS)], acc.at[pl.ds(base, ZROWS)])
    pltpu.sync_copy(vals_hbm.at[pl.ds(N, ZROWS)],
                    acc.at[pl.ds(base + ZROWS, ZROWS)])
    pltpu.sync_copy(vals_hbm.at[pl.ds(N, RPT - 2 * ZROWS)],
                    acc.at[pl.ds(base + 2 * ZROWS, RPT - 2 * ZROWS)])
    plsc.subcore_barrier()

    def step(j, carry):
      pltpu.sync_copy(vals_hbm.at[srcv.at[j]], rows)
      pltpu.sync_copy(rows, acc.at[dstv.at[j]], add=True)
      return carry

    lax.fori_loop(0, NBATCH, step, None)
    plsc.subcore_barrier()
    pltpu.sync_copy(acc.at[pl.ds(base, RPT)],
                    out_hbm.at[c, pl.ds(base, RPT)])

  return seg_kernel(values, src_idx, dst_idx)


_ROWBLK = 1280
_GRID = N_PAD // _ROWBLK


def _tc_prescale(deg_col, x):
  def body(deg_ref, x_ref, dinv_ref, xs_ref):
    dinv = lax.rsqrt(deg_ref[...])
    dinv_ref[...] = dinv
    xs_ref[...] = x_ref[...] * dinv

  return pl.pallas_call(
      body,
      grid=(_GRID,),
      in_specs=[
          pl.BlockSpec((_ROWBLK, 1), lambda i: (i, 0)),
          pl.BlockSpec((_ROWBLK, F), lambda i: (i, 0)),
      ],
      out_specs=[
          pl.BlockSpec((_ROWBLK, 1), lambda i: (i, 0)),
          pl.BlockSpec((_ROWBLK, F), lambda i: (i, 0)),
      ],
      out_shape=[
          jax.ShapeDtypeStruct((N_PAD, 1), jnp.float32),
          jax.ShapeDtypeStruct((N_PAD, F), jnp.float32),
      ],
  )(deg_col, x)


def _tc_layers(y, xs, dinv_col, w1, b1, w2p):
  def body(y0_ref, y1_ref, xs_ref, dinv_ref, w1_ref, b1_ref, w2_ref, ps_ref):
    dinv = dinv_ref[...]
    agg = (y0_ref[0] + y1_ref[0] + xs_ref[...]) * dinv
    h = jnp.dot(agg, w1_ref[...], preferred_element_type=jnp.float32)
    h = jnp.maximum(h + b1_ref[...], 0.0)
    p = jnp.dot(h, w2_ref[...], preferred_element_type=jnp.float32)
    rid = lax.broadcasted_iota(jnp.int32, (_ROWBLK, 1), 0)
    rid = rid + pl.program_id(0) * _ROWBLK
    ps_ref[...] = jnp.where(rid < N, p * dinv, 0.0)

  return pl.pallas_call(
      body,
      grid=(_GRID,),
      in_specs=[
          pl.BlockSpec((1, _ROWBLK, F), lambda i: (0, i, 0)),
          pl.BlockSpec((1, _ROWBLK, F), lambda i: (1, i, 0)),
          pl.BlockSpec((_ROWBLK, F), lambda i: (i, 0)),
          pl.BlockSpec((_ROWBLK, 1), lambda i: (i, 0)),
          pl.BlockSpec((F, HID), lambda i: (0, 0)),
          pl.BlockSpec((1, HID), lambda i: (0, 0)),
          pl.BlockSpec((HID, CPAD), lambda i: (0, 0)),
      ],
      out_specs=pl.BlockSpec((_ROWBLK, CPAD), lambda i: (i, 0)),
      out_shape=jax.ShapeDtypeStruct((N_PAD, CPAD), jnp.float32),
  )(y, y, xs, dinv_col, w1, b1, w2p)


def _tc_softmax(y2, ps, dinv_col, b2p):
  def body(y0_ref, y1_ref, ps_ref, dinv_ref, b2_ref, out_ref):
    z = (y0_ref[0] + y1_ref[0] + ps_ref[...]) * dinv_ref[...] + b2_ref[...]
    z0 = z[:, 0:1]
    z1 = z[:, 1:2]
    m = jnp.maximum(z0, z1)
    e0 = jnp.exp(z0 - m)
    e1 = jnp.exp(z1 - m)
    inv = 1.0 / (e0 + e1)
    out_ref[...] = jnp.concatenate([e0 * inv, e1 * inv], axis=1)

  return pl.pallas_call(
      body,
      grid=(_GRID,),
      in_specs=[
          pl.BlockSpec((1, _ROWBLK, CPAD), lambda i: (0, i, 0)),
          pl.BlockSpec((1, _ROWBLK, CPAD), lambda i: (1, i, 0)),
          pl.BlockSpec((_ROWBLK, CPAD), lambda i: (i, 0)),
          pl.BlockSpec((_ROWBLK, 1), lambda i: (i, 0)),
          pl.BlockSpec((1, CPAD), lambda i: (0, 0)),
      ],
      out_specs=pl.BlockSpec((_ROWBLK, 2), lambda i: (i, 0)),
      out_shape=jax.ShapeDtypeStruct((N_PAD, 2), jnp.float32),
  )(y2, y2, ps, dinv_col, b2p)


def kernel(X, edge_index, W1, b1, W2, b2):
  src = edge_index[0].astype(jnp.int32)
  dst = edge_index[1].astype(jnp.int32)
  pad = jnp.full((E_PAD - E,), N, jnp.int32)
  srcp = jnp.concatenate([src, pad]).reshape(NTILES, NBATCH, BATCH)
  dstp = jnp.concatenate([dst, pad]).reshape(NTILES, NBATCH, BATCH)
  xp = jnp.concatenate([X, jnp.zeros((N_PAD - N, F), X.dtype)], axis=0)

  degpart = _sc_degree(dstp)
  deg_col = (degpart[0] + degpart[1] + 1.0)[:, None]
  dinv_col, xs = _tc_prescale(deg_col, xp)

  y = _sc_seg_sum(xs, srcp, dstp, F)

  w2p = jnp.pad(W2, ((0, 0), (0, CPAD - W2.shape[1])))
  b2p = jnp.pad(b2, (0, CPAD - b2.shape[0]))[None, :]
  ps = _tc_layers(y, xs, dinv_col, W1, b1[None, :], w2p)

  y2 = _sc_seg_sum(ps, srcp, dstp, CPAD)
  out = _tc_softmax(y2, ps, dinv_col, b2p)
  return out[:N]
